# Initial kernel scaffold; baseline (speedup 1.0000x reference)
import jax
import jax.numpy as jnp
from jax.experimental import pallas as pl


def kernel(features, edge_index, W, b):
    raise NotImplementedError("write your pallas kernel here")


# speedup vs baseline: 2.7807x; 2.7807x over previous
import jax
import jax.numpy as jnp
from jax import lax
from jax.experimental import pallas as pl
from jax.experimental.pallas import tpu as pltpu
from jax.experimental.pallas import tpu_sc as plsc

F32 = jnp.float32

_NC = 2
_NS = 16
_CH = 128
_RB = 1024


def _pad_up(x, m):
    return ((x + m - 1) // m) * m


def kernel(features, edge_index, W, b):
    N, D = features.shape
    Dout = W.shape[0]
    E = edge_index.shape[1]
    DH = D // _NC

    NP = _pad_up(N, _RB)
    RPS = NP // _NS
    EP = _pad_up(E, _NC * _NS * _CH)
    NCH = EP // (_NS * _CH)
    NCHD = EP // (_NC * _NS * _CH)
    NB = NP // _RB

    src = edge_index[0]
    dst = edge_index[1]
    pad_e = EP - E
    srcp = jnp.concatenate([src, jnp.zeros((pad_e,), jnp.int32)])
    dstp = jnp.concatenate([dst, jnp.full((pad_e,), NP - 1, jnp.int32)])
    src_off = jnp.stack([srcp, srcp + NP]).reshape(_NC, _NS, NCH, _CH)
    dst_g = dstp.reshape(_NS, NCH, _CH)
    dst_d = dstp.reshape(_NC, _NS, NCHD, _CH)
    f_pad = jnp.pad(features, ((0, NP - N), (0, 0)))

    mesh = plsc.VectorSubcoreMesh(core_axis_name="c", subcore_axis_name="s")

    def _deg_body(dst_hbm, deg_hbm, idx_v, ones_v, zero_v, acc_sh):
        c = lax.axis_index("c")
        s = lax.axis_index("s")

        def fill_z(i, carry):
            zero_v[pl.ds(i * 16, 16)] = jnp.zeros((16,), F32)
            return carry

        lax.fori_loop(0, RPS // 16, fill_z, 0)
        for q in range(_CH // 16):
            ones_v[pl.ds(q * 16, 16)] = jnp.ones((16,), F32)
        pltpu.sync_copy(zero_v, acc_sh.at[pl.ds(s * RPS, RPS)])
        plsc.subcore_barrier()

        pltpu.sync_copy(dst_hbm.at[c, s], idx_v)

        def count(j, carry):
            pltpu.sync_copy(ones_v, acc_sh.at[idx_v.at[j]], add=True)
            return carry

        lax.fori_loop(0, NCHD, count, 0)
        plsc.subcore_barrier()
        pltpu.sync_copy(acc_sh.at[pl.ds(s * RPS, RPS)],
                        deg_hbm.at[c, pl.ds(s * RPS, RPS)])

    deg2 = pl.kernel(
        _deg_body,
        out_type=jax.ShapeDtypeStruct((_NC, NP), F32),
        mesh=mesh,
        scratch_types=[
            pltpu.VMEM((NCHD, _CH), jnp.int32),
            pltpu.VMEM((_CH,), F32),
            pltpu.VMEM((RPS,), F32),
            pltpu.VMEM_SHARED((NP,), F32),
        ],
    )(dst_d)

    def _norm_body(deg_ref, norm_ref):
        d = deg_ref[0:1, :] + deg_ref[1:2, :]
        norm_ref[...] = jnp.where(d > 0, lax.rsqrt(d), 0.0)

    norm_row = pl.pallas_call(
        _norm_body,
        grid=(1,),
        in_specs=[pl.BlockSpec((_NC, NP), lambda i: (0, 0))],
        out_specs=pl.BlockSpec((1, NP), lambda i: (0, 0)),
        out_shape=jax.ShapeDtypeStruct((1, NP), F32),
    )(deg2)
    norm_col = norm_row.reshape(NP, 1)

    def _scale_body(f_ref, n_ref, h_ref):
        h_ref[...] = f_ref[...] * n_ref[...]

    h_split = pl.pallas_call(
        _scale_body,
        grid=(_NC, NB),
        in_specs=[
            pl.BlockSpec((_RB, DH), lambda c, i: (i, c)),
            pl.BlockSpec((_RB, 1), lambda c, i: (i, 0)),
        ],
        out_specs=pl.BlockSpec((_RB, DH), lambda c, i: (c * NB + i, 0)),
        out_shape=jax.ShapeDtypeStruct((_NC * NP, DH), F32),
    )(f_pad, norm_col)

    def _scatter_body(h_hbm, src_hbm, dstg_hbm, agg_hbm,
                      src_v, dst_v, rows_v, acc_sh):
        c = lax.axis_index("c")
        s = lax.axis_index("s")

        def fill_z(i, carry):
            for q in range(DH // 16):
                rows_v[i, pl.ds(q * 16, 16)] = jnp.zeros((16,), F32)
            return carry

        lax.fori_loop(0, _CH, fill_z, 0)

        def zcopy(k, carry):
            pltpu.sync_copy(rows_v, acc_sh.at[pl.ds(s * RPS + k * _CH, _CH)])
            return carry

        lax.fori_loop(0, RPS // _CH, zcopy, 0)
        plsc.subcore_barrier()

        pltpu.sync_copy(src_hbm.at[c, s], src_v)
        pltpu.sync_copy(dstg_hbm.at[s], dst_v)

        def step(j, carry):
            pltpu.sync_copy(h_hbm.at[src_v.at[j]], rows_v)
            pltpu.sync_copy(rows_v, acc_sh.at[dst_v.at[j]], add=True)
            return carry

        lax.fori_loop(0, NCH, step, 0)
        plsc.subcore_barrier()
        pltpu.sync_copy(acc_sh.at[pl.ds(s * RPS, RPS)],
                        agg_hbm.at[pl.ds(c * NP + s * RPS, RPS)])

    agg = pl.kernel(
        _scatter_body,
        out_type=jax.ShapeDtypeStruct((_NC * NP, DH), F32),
        mesh=mesh,
        scratch_types=[
            pltpu.VMEM((NCH, _CH), jnp.int32),
            pltpu.VMEM((NCH, _CH), jnp.int32),
            pltpu.VMEM((_CH, DH), F32),
            pltpu.VMEM_SHARED((NP, DH), F32),
        ],
    )(h_split, src_off, dst_g)

    def _final_body(a_l, a_r, n_ref, w_ref, b_ref, o_ref):
        agg_blk = jnp.concatenate([a_l[...], a_r[...]], axis=1)
        h2 = agg_blk * n_ref[...]
        o_ref[...] = lax.dot_general(
            h2, w_ref[...], (((1,), (1,)), ((), ())),
            preferred_element_type=F32) + b_ref[...]

    out_pad = pl.pallas_call(
        _final_body,
        grid=(NB,),
        in_specs=[
            pl.BlockSpec((_RB, DH), lambda i: (i, 0)),
            pl.BlockSpec((_RB, DH), lambda i: (NB + i, 0)),
            pl.BlockSpec((_RB, 1), lambda i: (i, 0)),
            pl.BlockSpec((Dout, D), lambda i: (0, 0)),
            pl.BlockSpec((1, Dout), lambda i: (0, 0)),
        ],
        out_specs=pl.BlockSpec((_RB, Dout), lambda i: (i, 0)),
        out_shape=jax.ShapeDtypeStruct((NP, Dout), F32),
    )(agg, agg, norm_col, W, b.reshape(1, Dout))

    return out_pad[:N]
